# Initial kernel scaffold; baseline (speedup 1.0000x reference)
import jax
import jax.numpy as jnp
from jax.experimental import pallas as pl


def kernel(inputs, targets):
    raise NotImplementedError("write your pallas kernel here")


# speedup vs baseline: 14.2093x; 14.2093x over previous
import functools

import jax
import jax.numpy as jnp
from jax.experimental import pallas as pl
from jax.experimental.pallas import tpu as pltpu

BINS = 10
ROWS = 16384
COLS = 1024
BLOCK_ROWS = 512


def _hist_kernel(x_ref, t_ref, ge_ref):
    i = pl.program_id(0)

    @pl.when(i == 0)
    def _init():
        for k in range(BINS):
            ge_ref[0, k] = 0

    p = jax.nn.sigmoid(x_ref[...])
    g10 = jnp.abs(p - t_ref[...]) * BINS
    for k in range(1, BINS):
        ge_ref[0, k] += jnp.sum((g10 >= k).astype(jnp.int32))


def _loss_kernel(ge_ref, x_ref, t_ref, out_ref, *, tot):
    ge = [jnp.int32(tot)] + [ge_ref[0, k] for k in range(1, BINS)] + [jnp.int32(0)]
    counts = [ge[k] - ge[k + 1] for k in range(BINS)]
    n = functools.reduce(
        lambda a, b: a + b, [(c > 0).astype(jnp.int32) for c in counts]
    )
    nf = n.astype(jnp.float32)
    coef = [
        jnp.where(c > 0, 1.0 / (c.astype(jnp.float32) * nf), 0.0) for c in counts
    ]

    x = x_ref[...]
    t = t_ref[...]
    p = jax.nn.sigmoid(x)
    g10 = jnp.abs(p - t) * BINS
    w = jnp.full(x.shape, coef[0])
    for k in range(1, BINS):
        w += (coef[k] - coef[k - 1]) * (g10 >= k).astype(jnp.float32)
    bce = jnp.maximum(x, 0.0) - x * t + jnp.log1p(jnp.exp(-jnp.abs(x)))
    out_ref[...] = w * bce


def kernel(inputs, targets):
    rows, cols = inputs.shape
    tot = rows * cols
    grid = (rows // BLOCK_ROWS,)
    data_spec = pl.BlockSpec((BLOCK_ROWS, cols), lambda i: (i, 0))

    ge = pl.pallas_call(
        _hist_kernel,
        grid=grid,
        in_specs=[data_spec, data_spec],
        out_specs=pl.BlockSpec(memory_space=pltpu.SMEM),
        out_shape=jax.ShapeDtypeStruct((1, BINS), jnp.int32),
        compiler_params=pltpu.CompilerParams(
            dimension_semantics=("arbitrary",),
        ),
    )(inputs, targets)

    loss = pl.pallas_call(
        functools.partial(_loss_kernel, tot=tot),
        grid=grid,
        in_specs=[
            pl.BlockSpec(memory_space=pltpu.SMEM),
            data_spec,
            data_spec,
        ],
        out_specs=data_spec,
        out_shape=jax.ShapeDtypeStruct((rows, cols), jnp.float32),
        compiler_params=pltpu.CompilerParams(
            dimension_semantics=("parallel",),
        ),
    )(ge, inputs, targets)
    return loss
